# baseline (speedup 1.0000x reference)
import functools

import jax
import jax.numpy as jnp
from jax import lax
from jax.experimental import pallas as pl
from jax.experimental.pallas import tpu as pltpu
from jax.experimental.pallas import tpu_sc as plsc

N = 10000
F = 256
H = 4
NP = 10240
DN = H * NP
NC = 2
NS = 16
W = NC * NS
L = 16
GC = 64
SL = DN // NS
NEG = -3.0e38

_SC_PARAMS = pltpu.CompilerParams(use_tc_tiling_on_sc=False,
                                  needs_layout_passes=False,
                                  disable_bounds_checks=True)


def _mm_body(x_ref, wq_ref, wk_ref, bq_ref, bk_ref, q_ref, k_ref):
  xb = x_ref[...]
  q_ref[...] = jnp.dot(xb, wq_ref[...], preferred_element_type=jnp.float32) + bq_ref[...]
  k_ref[...] = jnp.dot(xb, wk_ref[...], preferred_element_type=jnp.float32) + bk_ref[...]


def _project_qk(x, Wq, Wk, bq, bk):
  blk = 1000
  grid = N // blk
  return pl.pallas_call(
      _mm_body,
      grid=(grid,),
      in_specs=[
          pl.BlockSpec((blk, F), lambda i: (i, 0)),
          pl.BlockSpec((F, H * F), lambda i: (0, 0)),
          pl.BlockSpec((F, H * F), lambda i: (0, 0)),
          pl.BlockSpec((1, H * F), lambda i: (0, 0)),
          pl.BlockSpec((1, H * F), lambda i: (0, 0)),
      ],
      out_specs=[
          pl.BlockSpec((blk, H * F), lambda i: (i, 0)),
          pl.BlockSpec((blk, H * F), lambda i: (i, 0)),
      ],
      out_shape=[
          jax.ShapeDtypeStruct((N, H * F), jnp.float32),
          jax.ShapeDtypeStruct((N, H * F), jnp.float32),
      ],
  )(x, Wq, Wk, bq, bk)


def _wid():
  return lax.axis_index("s") * NC + lax.axis_index("c")


def _seg_max_update(table, lidx, val, valid):
  cur0 = plsc.load_gather(table, [lidx])
  pend0 = valid & (cur0 < val)

  def cond(st):
    return jnp.any(st[0])

  def body(st):
    pend, want = st
    plsc.store_scatter(table, [lidx], want, mask=pend)
    cur = plsc.load_gather(table, [lidx])
    return (pend & (cur < want), want)

  lax.while_loop(cond, body, (pend0, val))


def _scores_kernel(E, EW, NCH, E_pad):
  mesh = plsc.VectorSubcoreMesh(
      core_axis_name="c", subcore_axis_name="s", num_cores=NC, num_subcores=NS)
  NG = NCH // GC
  GE = GC * L

  @functools.partial(
      pl.kernel,
      out_type=[
          jax.ShapeDtypeStruct((H, E_pad), jnp.float32),
          jax.ShapeDtypeStruct((NC, DN), jnp.float32),
      ],
      mesh=mesh,
      scratch_types=[
          pltpu.VMEM((2, L, H * F), jnp.float32),
          pltpu.VMEM((2, L, H * F), jnp.float32),
          pltpu.VMEM((GE,), jnp.int32),
          pltpu.VMEM((GE,), jnp.int32),
          pltpu.VMEM((H * GE,), jnp.float32),
          pltpu.VMEM((DN,), jnp.float32),
          pltpu.VMEM((SL,), jnp.float32),
          pltpu.VMEM((SL,), jnp.float32),
          pltpu.VMEM_SHARED((NS, SL), jnp.float32),
          pltpu.SemaphoreType.DMA,
          pltpu.SemaphoreType.DMA,
          pltpu.SemaphoreType.DMA,
          pltpu.SemaphoreType.DMA,
      ],
      compiler_params=_SC_PARAMS,
  )
  def k(q_hbm, k_hbm, row_hbm, col_hbm, s_hbm, mpart_hbm,
        qbuf, kbuf, idxr, idxc, sbuf, lm, mtmp, macc, shared,
        sq0, sq1, sk0, sk1):
    cid = lax.axis_index("c")
    sid = lax.axis_index("s")
    wid = sid * NC + cid
    base = wid * EW
    iota = lax.iota(jnp.int32, L)
    neg = jnp.full((L,), NEG, jnp.float32)
    semq = (sq0, sq1)
    semk = (sk0, sk1)

    def init_body(i, _):
      lm[pl.ds(i * L, L)] = neg
      return _
    lax.fori_loop(0, DN // L, init_body, None)

    def start(j, b):
      pltpu.async_copy(q_hbm.at[idxr.at[pl.ds(j * L, L)]], qbuf.at[b], semq[b])
      pltpu.async_copy(k_hbm.at[idxc.at[pl.ds(j * L, L)]], kbuf.at[b], semk[b])

    def wait(b):
      pltpu.make_async_copy(q_hbm.at[idxr.at[pl.ds(0, L)]], qbuf.at[b],
                            semq[b]).wait()
      pltpu.make_async_copy(k_hbm.at[idxc.at[pl.ds(0, L)]], kbuf.at[b],
                            semk[b]).wait()

    def compute(gb, j, b):
      rvec = idxr[pl.ds(j * L, L)]
      valid = (gb + j * L + iota) < E
      zero4 = tuple(jnp.zeros((L,), jnp.float32) for _ in range(H))

      def e_body(e2, carry):
        out = list(carry)
        for de in range(2):
          e = e2 * 2 + de
          for h in range(H):
            facc = jnp.zeros((L,), jnp.float32)
            for c in range(F // L):
              off = h * F + c * L
              facc = facc + qbuf[b, e, pl.ds(off, L)] * kbuf[b, e, pl.ds(off, L)]
            s = jnp.sum(facc)
            out[h] = jnp.where(iota == e, s, out[h])
        return tuple(out)

      svecs = lax.fori_loop(0, L // 2, e_body, zero4)
      for h in range(H):
        acc = svecs[h]
        sbuf[pl.ds(h * GE + j * L, L)] = acc
        lidx = jnp.full((L,), h * NP, jnp.int32) + rvec
        _seg_max_update(lm, lidx, acc, valid)

    def group_body(gi, _):
      gb = base + gi * GE
      pltpu.sync_copy(row_hbm.at[pl.ds(gb, GE)], idxr)
      pltpu.sync_copy(col_hbm.at[pl.ds(gb, GE)], idxc)
      start(0, 0)
      def pair_body(g2, _):
        j0 = g2 * 2
        start(j0 + 1, 1)
        wait(0)
        compute(gb, j0, 0)
        @pl.when(g2 < GC // 2 - 1)
        def _():
          start(j0 + 2, 0)
        wait(1)
        compute(gb, j0 + 1, 1)
        return _
      lax.fori_loop(0, GC // 2, pair_body, None)
      for h in range(H):
        pltpu.sync_copy(sbuf.at[pl.ds(h * GE, GE)],
                        s_hbm.at[h, pl.ds(gb, GE)])
      return _
    lax.fori_loop(0, NG, group_body, None)

    for r in range(NS):
      pltpu.sync_copy(lm.at[pl.ds(r * SL, SL)], shared.at[sid])
      plsc.subcore_barrier()
      @pl.when(sid == r)
      def _():
        pltpu.sync_copy(shared.at[0], macc)
        for t in range(1, NS):
          pltpu.sync_copy(shared.at[t], mtmp)
          def mx_body(j, _):
            macc[pl.ds(j * L, L)] = jnp.maximum(macc[pl.ds(j * L, L)],
                                                mtmp[pl.ds(j * L, L)])
            return _
          lax.fori_loop(0, SL // L, mx_body, None)
        pltpu.sync_copy(macc, mpart_hbm.at[cid, pl.ds(r * SL, SL)])
      plsc.subcore_barrier()

  return k


def _expden_kernel(E, EW, NCH, E_pad):
  mesh = plsc.VectorSubcoreMesh(
      core_axis_name="c", subcore_axis_name="s", num_cores=NC, num_subcores=NS)

  @functools.partial(
      pl.kernel,
      out_type=[
          jax.ShapeDtypeStruct((H, E_pad), jnp.float32),
          jax.ShapeDtypeStruct((NC, DN), jnp.float32),
      ],
      mesh=mesh,
      scratch_types=[
          pltpu.VMEM((DN,), jnp.float32),
          pltpu.VMEM((1024,), jnp.float32),
          pltpu.VMEM((SL,), jnp.float32),
          pltpu.VMEM((H * GC * L,), jnp.float32),
          pltpu.VMEM((H * GC * L,), jnp.float32),
          pltpu.VMEM((GC * L,), jnp.int32),
          pltpu.VMEM((H * GC * L // 128, 128), jnp.int32),
          pltpu.VMEM_SHARED((DN,), jnp.float32),
          pltpu.SemaphoreType.DMA,
      ],
      compiler_params=_SC_PARAMS,
  )
  def k(s_hbm, row_hbm, mpart_hbm, ex_hbm, dpart_hbm,
        ml, tmp, zbuf, sbg, exf, idxg, lidxg, dacc, sem):
    cid = lax.axis_index("c")
    sid = lax.axis_index("s")
    wid = sid * NC + cid
    base = wid * EW
    iota = lax.iota(jnp.int32, L)
    zero = jnp.zeros((L,), jnp.float32)
    GE = GC * L
    NBH = GE // 128

    def z_body(i, _):
      zbuf[pl.ds(i * L, L)] = zero
      return _
    lax.fori_loop(0, SL // L, z_body, None)
    pltpu.sync_copy(zbuf, dacc.at[pl.ds(sid * SL, SL)])
    plsc.subcore_barrier()

    pltpu.sync_copy(mpart_hbm.at[0], ml)
    def mg_body(c, _):
      pltpu.sync_copy(mpart_hbm.at[1, pl.ds(c * 1024, 1024)], tmp)
      def mx_body(j, _):
        o = c * 1024 + j * L
        ml[pl.ds(o, L)] = jnp.maximum(ml[pl.ds(o, L)], tmp[pl.ds(j * L, L)])
        return _
      lax.fori_loop(0, 1024 // L, mx_body, None)
      return _
    lax.fori_loop(0, DN // 1024, mg_body, None)

    def group_body(gi, _):
      gb = base + gi * GE
      pltpu.sync_copy(row_hbm.at[pl.ds(gb, GE)], idxg)
      for h in range(H):
        pltpu.sync_copy(s_hbm.at[h, pl.ds(gb, GE)],
                        sbg.at[pl.ds(h * GE, GE)])

      def chunk_body(j, _):
        rvec = idxg[pl.ds(j * L, L)]
        valid = (gb + j * L + iota) < E
        nb0 = j // 8
        lo = (j % 8) * L
        for h in range(H):
          sv = sbg[pl.ds(h * GE + j * L, L)]
          lidx = jnp.full((L,), h * NP, jnp.int32) + rvec
          mrow = plsc.load_gather(ml, [lidx])
          ex = jnp.where(valid, jnp.exp(sv - mrow), 0.0)
          exf[pl.ds(h * GE + j * L, L)] = ex
          lidxg[h * NBH + nb0, pl.ds(lo, L)] = lidx
        return _
      lax.fori_loop(0, GC, chunk_body, None)

      for h in range(H):
        pltpu.sync_copy(exf.at[pl.ds(h * GE, GE)],
                        ex_hbm.at[h, pl.ds(gb, GE)])
      for nb in range(H * NBH):
        pltpu.sync_copy(exf.at[pl.ds(nb * 128, 128)],
                        dacc.at[lidxg.at[nb]], add=True)
      return _
    lax.fori_loop(0, NCH // GC, group_body, None)

    plsc.subcore_barrier()
    @pl.when(sid == 0)
    def _():
      pltpu.sync_copy(dacc, dpart_hbm.at[cid])

  return k


def _norm_kernel(E, EW, NCH, E_pad):
  mesh = plsc.VectorSubcoreMesh(
      core_axis_name="c", subcore_axis_name="s", num_cores=NC, num_subcores=NS)

  @functools.partial(
      pl.kernel,
      out_type=jax.ShapeDtypeStruct((E_pad,), jnp.float32),
      mesh=mesh,
      scratch_types=[
          pltpu.VMEM((DN,), jnp.float32),
          pltpu.VMEM((1024,), jnp.float32),
          pltpu.VMEM((H * GC * L,), jnp.float32),
          pltpu.VMEM((GC * L,), jnp.int32),
          pltpu.VMEM((GC * L,), jnp.float32),
          pltpu.SemaphoreType.DMA,
      ],
      compiler_params=_SC_PARAMS,
  )
  def k(ex_hbm, row_hbm, dpart_hbm, vals_hbm, dl, tmp, exf, idxg, og, sem):
    cid = lax.axis_index("c")
    sid = lax.axis_index("s")
    wid = sid * NC + cid
    base = wid * EW
    iota = lax.iota(jnp.int32, L)
    GE = GC * L

    pltpu.sync_copy(dpart_hbm.at[0], dl)
    def dg_body(c, _):
      pltpu.sync_copy(dpart_hbm.at[1, pl.ds(c * 1024, 1024)], tmp)
      def ad_body(j, _):
        o = c * 1024 + j * L
        dl[pl.ds(o, L)] = dl[pl.ds(o, L)] + tmp[pl.ds(j * L, L)]
        return _
      lax.fori_loop(0, 1024 // L, ad_body, None)
      return _
    lax.fori_loop(0, DN // 1024, dg_body, None)

    def group_body(gi, _):
      gb = base + gi * GE
      pltpu.sync_copy(row_hbm.at[pl.ds(gb, GE)], idxg)
      for h in range(H):
        pltpu.sync_copy(ex_hbm.at[h, pl.ds(gb, GE)],
                        exf.at[pl.ds(h * GE, GE)])

      def chunk_body(j, _):
        rvec = idxg[pl.ds(j * L, L)]
        acc = jnp.zeros((L,), jnp.float32)
        for h in range(H):
          lidx = jnp.full((L,), h * NP, jnp.int32) + rvec
          dv = plsc.load_gather(dl, [lidx])
          acc = acc + exf[pl.ds(h * GE + j * L, L)] / dv
        og[pl.ds(j * L, L)] = acc * (1.0 / H)
        return _
      lax.fori_loop(0, GC, chunk_body, None)

      pltpu.sync_copy(og, vals_hbm.at[pl.ds(gb, GE)])
      return _
    lax.fori_loop(0, NCH // GC, group_body, None)

  return k


@jax.jit
def kernel(x, edge_index, W_qk, b_qk):
  E = edge_index.shape[1]
  NCH = -(-E // (W * L))
  NCH = -(-NCH // GC) * GC
  EW = NCH * L
  E_pad = W * EW

  W4 = W_qk.reshape(F, H, 2, F)
  Wq = W4[:, :, 0, :].reshape(F, H * F)
  Wk = W4[:, :, 1, :].reshape(F, H * F)
  b4 = b_qk.reshape(H, 2, F)
  bq = b4[:, 0, :].reshape(1, H * F)
  bk = b4[:, 1, :].reshape(1, H * F)

  q, k = _project_qk(x, Wq, Wk, bq, bk)

  row = jnp.pad(edge_index[0], (0, E_pad - E))
  col = jnp.pad(edge_index[1], (0, E_pad - E))

  s, mpart = _scores_kernel(E, EW, NCH, E_pad)(q, k, row, col)
  ex, dpart = _expden_kernel(E, EW, NCH, E_pad)(s, row, mpart)
  vals = _norm_kernel(E, EW, NCH, E_pad)(ex, row, dpart)
  return vals[:E]


# speedup vs baseline: 1.4191x; 1.4191x over previous
import functools

import jax
import jax.numpy as jnp
from jax import lax
from jax.experimental import pallas as pl
from jax.experimental.pallas import tpu as pltpu
from jax.experimental.pallas import tpu_sc as plsc

N = 10000
F = 256
H = 4
NP = 10240
DN = H * NP
NC = 2
NS = 16
W = NC * NS
L = 16
GC = 64
SL = DN // NS
NEG = -3.0e38

_SC_PARAMS = pltpu.CompilerParams(use_tc_tiling_on_sc=False,
                                  needs_layout_passes=False,
                                  disable_bounds_checks=True)


def _mm_body(x_ref, wq_ref, wk_ref, bq_ref, bk_ref, q_ref, k_ref):
  xb = x_ref[...]
  q_ref[...] = jnp.dot(xb, wq_ref[...], preferred_element_type=jnp.float32) + bq_ref[...]
  k_ref[...] = jnp.dot(xb, wk_ref[...], preferred_element_type=jnp.float32) + bk_ref[...]


def _project_qk(x, Wq, Wk, bq, bk):
  blk = 1000
  grid = N // blk
  return pl.pallas_call(
      _mm_body,
      grid=(grid,),
      in_specs=[
          pl.BlockSpec((blk, F), lambda i: (i, 0)),
          pl.BlockSpec((F, H * F), lambda i: (0, 0)),
          pl.BlockSpec((F, H * F), lambda i: (0, 0)),
          pl.BlockSpec((1, H * F), lambda i: (0, 0)),
          pl.BlockSpec((1, H * F), lambda i: (0, 0)),
      ],
      out_specs=[
          pl.BlockSpec((blk, H * F), lambda i: (i, 0)),
          pl.BlockSpec((blk, H * F), lambda i: (i, 0)),
      ],
      out_shape=[
          jax.ShapeDtypeStruct((N, H * F), jnp.float32),
          jax.ShapeDtypeStruct((N, H * F), jnp.float32),
      ],
  )(x, Wq, Wk, bq, bk)


def _wid():
  return lax.axis_index("s") * NC + lax.axis_index("c")


def _seg_max_update(table, lidx, val, valid):
  cur0 = plsc.load_gather(table, [lidx])
  pend0 = valid & (cur0 < val)

  def cond(st):
    return jnp.any(st[0])

  def body(st):
    pend, want = st
    plsc.store_scatter(table, [lidx], want, mask=pend)
    cur = plsc.load_gather(table, [lidx])
    return (pend & (cur < want), want)

  lax.while_loop(cond, body, (pend0, val))


def _scores_kernel(E, EW, NCH, E_pad):
  mesh = plsc.VectorSubcoreMesh(
      core_axis_name="c", subcore_axis_name="s", num_cores=NC, num_subcores=NS)
  NG = NCH // GC
  GE = GC * L

  @functools.partial(
      pl.kernel,
      out_type=[
          jax.ShapeDtypeStruct((H, E_pad), jnp.float32),
          jax.ShapeDtypeStruct((NC, DN), jnp.float32),
      ],
      mesh=mesh,
      scratch_types=[
          pltpu.VMEM((2, L, H * F), jnp.float32),
          pltpu.VMEM((2, L, H * F), jnp.float32),
          pltpu.VMEM((GE,), jnp.int32),
          pltpu.VMEM((GE,), jnp.int32),
          pltpu.VMEM((H * GE,), jnp.float32),
          pltpu.VMEM((DN,), jnp.float32),
          pltpu.VMEM((SL,), jnp.float32),
          pltpu.VMEM((SL,), jnp.float32),
          pltpu.VMEM_SHARED((NS, SL), jnp.float32),
          pltpu.SemaphoreType.DMA,
          pltpu.SemaphoreType.DMA,
          pltpu.SemaphoreType.DMA,
          pltpu.SemaphoreType.DMA,
      ],
      compiler_params=_SC_PARAMS,
  )
  def k(q_hbm, k_hbm, row_hbm, col_hbm, s_hbm, mpart_hbm,
        qbuf, kbuf, idxr, idxc, sbuf, lm, mtmp, macc, shared,
        sq0, sq1, sk0, sk1):
    cid = lax.axis_index("c")
    sid = lax.axis_index("s")
    wid = sid * NC + cid
    base = wid * EW
    iota = lax.iota(jnp.int32, L)
    neg = jnp.full((L,), NEG, jnp.float32)
    semq = (sq0, sq1)
    semk = (sk0, sk1)

    def init_body(i, _):
      lm[pl.ds(i * L, L)] = neg
      return _
    lax.fori_loop(0, DN // L, init_body, None)

    def start(j, b):
      pltpu.async_copy(q_hbm.at[idxr.at[pl.ds(j * L, L)]], qbuf.at[b], semq[b])
      pltpu.async_copy(k_hbm.at[idxc.at[pl.ds(j * L, L)]], kbuf.at[b], semk[b])

    def wait(b):
      pltpu.make_async_copy(q_hbm.at[idxr.at[pl.ds(0, L)]], qbuf.at[b],
                            semq[b]).wait()
      pltpu.make_async_copy(k_hbm.at[idxc.at[pl.ds(0, L)]], kbuf.at[b],
                            semk[b]).wait()

    def compute(gb, j, b):
      rvec = idxr[pl.ds(j * L, L)]
      valid = (gb + j * L + iota) < E
      zero4 = tuple(jnp.zeros((L,), jnp.float32) for _ in range(H))

      def e_body(e, carry):
        out = []
        for h in range(H):
          facc = jnp.zeros((L,), jnp.float32)
          for c in range(F // L):
            off = h * F + c * L
            facc = facc + qbuf[b, e, pl.ds(off, L)] * kbuf[b, e, pl.ds(off, L)]
          s = jnp.sum(facc)
          out.append(jnp.where(iota == e, s, carry[h]))
        return tuple(out)

      svecs = lax.fori_loop(0, L, e_body, zero4)
      for h in range(H):
        acc = svecs[h]
        sbuf[pl.ds(h * GE + j * L, L)] = acc
        lidx = jnp.full((L,), h * NP, jnp.int32) + rvec
        _seg_max_update(lm, lidx, acc, valid)

    def group_body(gi, _):
      gb = base + gi * GE
      pltpu.sync_copy(row_hbm.at[pl.ds(gb, GE)], idxr)
      pltpu.sync_copy(col_hbm.at[pl.ds(gb, GE)], idxc)
      start(0, 0)
      def pair_body(g2, _):
        j0 = g2 * 2
        start(j0 + 1, 1)
        wait(0)
        compute(gb, j0, 0)
        @pl.when(g2 < GC // 2 - 1)
        def _():
          start(j0 + 2, 0)
        wait(1)
        compute(gb, j0 + 1, 1)
        return _
      lax.fori_loop(0, GC // 2, pair_body, None)
      for h in range(H):
        pltpu.sync_copy(sbuf.at[pl.ds(h * GE, GE)],
                        s_hbm.at[h, pl.ds(gb, GE)])
      return _
    lax.fori_loop(0, NG, group_body, None)

    for r in range(NS):
      pltpu.sync_copy(lm.at[pl.ds(r * SL, SL)], shared.at[sid])
      plsc.subcore_barrier()
      @pl.when(sid == r)
      def _():
        pltpu.sync_copy(shared.at[0], macc)
        for t in range(1, NS):
          pltpu.sync_copy(shared.at[t], mtmp)
          def mx_body(j, _):
            macc[pl.ds(j * L, L)] = jnp.maximum(macc[pl.ds(j * L, L)],
                                                mtmp[pl.ds(j * L, L)])
            return _
          lax.fori_loop(0, SL // L, mx_body, None)
        pltpu.sync_copy(macc, mpart_hbm.at[cid, pl.ds(r * SL, SL)])
      plsc.subcore_barrier()

  return k


def _expden_kernel(E, EW, NCH, E_pad):
  mesh = plsc.VectorSubcoreMesh(
      core_axis_name="c", subcore_axis_name="s", num_cores=NC, num_subcores=NS)

  @functools.partial(
      pl.kernel,
      out_type=[
          jax.ShapeDtypeStruct((H, E_pad), jnp.float32),
          jax.ShapeDtypeStruct((NC, DN), jnp.float32),
      ],
      mesh=mesh,
      scratch_types=[
          pltpu.VMEM((DN,), jnp.float32),
          pltpu.VMEM((1024,), jnp.float32),
          pltpu.VMEM((SL,), jnp.float32),
          pltpu.VMEM((H * GC * L,), jnp.float32),
          pltpu.VMEM((H * GC * L,), jnp.float32),
          pltpu.VMEM((GC * L,), jnp.int32),
          pltpu.VMEM((H * GC * L // 128, 128), jnp.int32),
          pltpu.VMEM_SHARED((DN,), jnp.float32),
          pltpu.SemaphoreType.DMA,
      ],
      compiler_params=_SC_PARAMS,
  )
  def k(s_hbm, row_hbm, mpart_hbm, ex_hbm, dpart_hbm,
        ml, tmp, zbuf, sbg, exf, idxg, lidxg, dacc, sem):
    cid = lax.axis_index("c")
    sid = lax.axis_index("s")
    wid = sid * NC + cid
    base = wid * EW
    iota = lax.iota(jnp.int32, L)
    zero = jnp.zeros((L,), jnp.float32)
    GE = GC * L
    NBH = GE // 128

    def z_body(i, _):
      zbuf[pl.ds(i * L, L)] = zero
      return _
    lax.fori_loop(0, SL // L, z_body, None)
    pltpu.sync_copy(zbuf, dacc.at[pl.ds(sid * SL, SL)])
    plsc.subcore_barrier()

    pltpu.sync_copy(mpart_hbm.at[0], ml)
    def mg_body(c, _):
      pltpu.sync_copy(mpart_hbm.at[1, pl.ds(c * 1024, 1024)], tmp)
      def mx_body(j, _):
        o = c * 1024 + j * L
        ml[pl.ds(o, L)] = jnp.maximum(ml[pl.ds(o, L)], tmp[pl.ds(j * L, L)])
        return _
      lax.fori_loop(0, 1024 // L, mx_body, None)
      return _
    lax.fori_loop(0, DN // 1024, mg_body, None)

    def group_body(gi, _):
      gb = base + gi * GE
      pltpu.sync_copy(row_hbm.at[pl.ds(gb, GE)], idxg)
      for h in range(H):
        pltpu.sync_copy(s_hbm.at[h, pl.ds(gb, GE)],
                        sbg.at[pl.ds(h * GE, GE)])

      def chunk_body(j, _):
        rvec = idxg[pl.ds(j * L, L)]
        valid = (gb + j * L + iota) < E
        nb0 = j // 8
        lo = (j % 8) * L
        for h in range(H):
          sv = sbg[pl.ds(h * GE + j * L, L)]
          lidx = jnp.full((L,), h * NP, jnp.int32) + rvec
          mrow = plsc.load_gather(ml, [lidx])
          ex = jnp.where(valid, jnp.exp(sv - mrow), 0.0)
          exf[pl.ds(h * GE + j * L, L)] = ex
          lidxg[h * NBH + nb0, pl.ds(lo, L)] = lidx
        return _
      lax.fori_loop(0, GC, chunk_body, None)

      for h in range(H):
        pltpu.sync_copy(exf.at[pl.ds(h * GE, GE)],
                        ex_hbm.at[h, pl.ds(gb, GE)])
      for nb in range(H * NBH):
        pltpu.sync_copy(exf.at[pl.ds(nb * 128, 128)],
                        dacc.at[lidxg.at[nb]], add=True)
      return _
    lax.fori_loop(0, NCH // GC, group_body, None)

    plsc.subcore_barrier()
    @pl.when(sid == 0)
    def _():
      pltpu.sync_copy(dacc, dpart_hbm.at[cid])

  return k


def _norm_kernel(E, EW, NCH, E_pad):
  mesh = plsc.VectorSubcoreMesh(
      core_axis_name="c", subcore_axis_name="s", num_cores=NC, num_subcores=NS)

  @functools.partial(
      pl.kernel,
      out_type=jax.ShapeDtypeStruct((E_pad,), jnp.float32),
      mesh=mesh,
      scratch_types=[
          pltpu.VMEM((DN,), jnp.float32),
          pltpu.VMEM((1024,), jnp.float32),
          pltpu.VMEM((H * GC * L,), jnp.float32),
          pltpu.VMEM((GC * L,), jnp.int32),
          pltpu.VMEM((GC * L,), jnp.float32),
          pltpu.SemaphoreType.DMA,
      ],
      compiler_params=_SC_PARAMS,
  )
  def k(ex_hbm, row_hbm, dpart_hbm, vals_hbm, dl, tmp, exf, idxg, og, sem):
    cid = lax.axis_index("c")
    sid = lax.axis_index("s")
    wid = sid * NC + cid
    base = wid * EW
    iota = lax.iota(jnp.int32, L)
    GE = GC * L

    pltpu.sync_copy(dpart_hbm.at[0], dl)
    def dg_body(c, _):
      pltpu.sync_copy(dpart_hbm.at[1, pl.ds(c * 1024, 1024)], tmp)
      def ad_body(j, _):
        o = c * 1024 + j * L
        dl[pl.ds(o, L)] = dl[pl.ds(o, L)] + tmp[pl.ds(j * L, L)]
        return _
      lax.fori_loop(0, 1024 // L, ad_body, None)
      return _
    lax.fori_loop(0, DN // 1024, dg_body, None)

    def group_body(gi, _):
      gb = base + gi * GE
      pltpu.sync_copy(row_hbm.at[pl.ds(gb, GE)], idxg)
      for h in range(H):
        pltpu.sync_copy(ex_hbm.at[h, pl.ds(gb, GE)],
                        exf.at[pl.ds(h * GE, GE)])

      def chunk_body(j, _):
        rvec = idxg[pl.ds(j * L, L)]
        acc = jnp.zeros((L,), jnp.float32)
        for h in range(H):
          lidx = jnp.full((L,), h * NP, jnp.int32) + rvec
          dv = plsc.load_gather(dl, [lidx])
          acc = acc + exf[pl.ds(h * GE + j * L, L)] / dv
        og[pl.ds(j * L, L)] = acc * (1.0 / H)
        return _
      lax.fori_loop(0, GC, chunk_body, None)

      pltpu.sync_copy(og, vals_hbm.at[pl.ds(gb, GE)])
      return _
    lax.fori_loop(0, NCH // GC, group_body, None)

  return k


@jax.jit
def kernel(x, edge_index, W_qk, b_qk):
  E = edge_index.shape[1]
  NCH = -(-E // (W * L))
  NCH = -(-NCH // GC) * GC
  EW = NCH * L
  E_pad = W * EW

  W4 = W_qk.reshape(F, H, 2, F)
  Wq = W4[:, :, 0, :].reshape(F, H * F)
  Wk = W4[:, :, 1, :].reshape(F, H * F)
  b4 = b_qk.reshape(H, 2, F)
  bq = b4[:, 0, :].reshape(1, H * F)
  bk = b4[:, 1, :].reshape(1, H * F)

  q, k = _project_qk(x, Wq, Wk, bq, bk)

  row = jnp.pad(edge_index[0], (0, E_pad - E))
  col = jnp.pad(edge_index[1], (0, E_pad - E))

  s, mpart = _scores_kernel(E, EW, NCH, E_pad)(q, k, row, col)
  ex, dpart = _expden_kernel(E, EW, NCH, E_pad)(s, row, mpart)
  vals = _norm_kernel(E, EW, NCH, E_pad)(ex, row, dpart)
  return vals[:E]
